# baseline (speedup 1.0000x reference)
import functools

import jax
import jax.numpy as jnp
from jax import lax
from jax.experimental import pallas as pl
from jax.experimental.pallas import tpu as pltpu
from jax.experimental.pallas import tpu_sc as plsc

NC, NS, L = 2, 16, 16
B, N, D = 8, 4096, 3
NW = NC * NS
WQ = N // NW
QB = 8
NV = N // L

BT = B - 1
BLKN = 512
NBLK = N // BLKN


def _sc_body(x1f, x2f, x1r, x2r, out1, outc,
             q1x, q1y, q1z, x2x, x2y, x2z, hq2, hp2, colmin, rowacc,
             tbuf, ovec, shared, dsem):
    c = lax.axis_index("c")
    s = lax.axis_index("s")
    wid = c * NS + s

    qb0 = wid * WQ
    cps = [pltpu.async_copy(x2f.at[pl.ds(0, N)], x2x, dsem),
           pltpu.async_copy(x2f.at[pl.ds(N, N)], x2y, dsem),
           pltpu.async_copy(x2f.at[pl.ds(2 * N, N)], x2z, dsem),
           pltpu.async_copy(x1f.at[pl.ds(qb0, WQ)], q1x, dsem),
           pltpu.async_copy(x1f.at[pl.ds(N + qb0, WQ)], q1y, dsem),
           pltpu.async_copy(x1f.at[pl.ds(2 * N + qb0, WQ)], q1z, dsem)]
    for cp in cps:
        cp.wait()

    half = jnp.float32(0.5)

    def hp_body(j, carry):
        sl = pl.ds(j * L, L)
        px, py, pz = x2x[sl], x2y[sl], x2z[sl]
        hp2[sl] = (px * px + py * py + pz * pz) * half
        return carry
    lax.fori_loop(0, NV, hp_body, 0)

    def hq_body(j, carry):
        sl = pl.ds(j * L, L)
        qx, qy, qz = q1x[sl], q1y[sl], q1z[sl]
        hq2[sl] = (qx * qx + qy * qy + qz * qz) * half
        return carry
    lax.fori_loop(0, WQ // L, hq_body, 0)

    cps = [pltpu.async_copy(x2r.at[pl.ds(0, N)], x2x, dsem),
           pltpu.async_copy(x2r.at[pl.ds(N, N)], x2y, dsem),
           pltpu.async_copy(x2r.at[pl.ds(2 * N, N)], x2z, dsem),
           pltpu.async_copy(x1r.at[pl.ds(qb0, WQ)], q1x, dsem),
           pltpu.async_copy(x1r.at[pl.ds(N + qb0, WQ)], q1y, dsem),
           pltpu.async_copy(x1r.at[pl.ds(2 * N + qb0, WQ)], q1z, dsem)]
    for cp in cps:
        cp.wait()

    inf_v = jnp.full((L,), jnp.inf, jnp.float32)

    def init_body(j, carry):
        colmin[pl.ds(j * L, L)] = inf_v
        return carry
    lax.fori_loop(0, NV, init_body, 0)

    def qg_body(qg, carry):
        sl = pl.ds(qg * L, L)
        qvx, qvy, qvz, qv2 = q1x[sl], q1y[sl], q1z[sl], hq2[sl]
        for h in range(L // QB):
            qx = [jnp.full((L,), qvx[h * QB + i]) for i in range(QB)]
            qy = [jnp.full((L,), qvy[h * QB + i]) for i in range(QB)]
            qz = [jnp.full((L,), qvz[h * QB + i]) for i in range(QB)]
            q2 = [jnp.full((L,), qv2[h * QB + i]) for i in range(QB)]

            def j_body(j, accs):
                jsl = pl.ds(j * L, L)
                px = x2x[jsl]
                py = x2y[jsl]
                pz = x2z[jsl]
                p2 = hp2[jsl]
                ds_ = []
                new_accs = []
                for i in range(QB):
                    cr = px * qx[i] + py * qy[i] + pz * qz[i]
                    d = (p2 + q2[i]) - cr
                    ds_.append(d)
                    new_accs.append(jnp.minimum(accs[i], d))
                m01 = jnp.minimum(ds_[0], ds_[1])
                m23 = jnp.minimum(ds_[2], ds_[3])
                m45 = jnp.minimum(ds_[4], ds_[5])
                m67 = jnp.minimum(ds_[6], ds_[7])
                m = jnp.minimum(jnp.minimum(m01, m23),
                                jnp.minimum(m45, m67))
                colmin[jsl] = jnp.minimum(colmin[jsl], m)
                return tuple(new_accs)

            accs = lax.fori_loop(0, NV, j_body, (inf_v,) * QB)
            for i in range(QB):
                q = qg * L + h * QB + i
                rowacc[pl.ds(q * L, L)] = accs[i]
        return carry

    lax.fori_loop(0, WQ // L, qg_body, 0)

    iota = lax.iota(jnp.int32, L)

    def rg_body(g, s1v):
        idx0 = g * (L * L) + iota * L
        rmin = plsc.load_gather(rowacc, [idx0])
        for j in range(1, L):
            rmin = jnp.minimum(rmin, plsc.load_gather(rowacc, [idx0 + j]))
        return s1v + rmin

    s1v = lax.fori_loop(0, WQ // L, rg_body, jnp.zeros((L,), jnp.float32))
    ovec[pl.ds(0, L)] = s1v
    pltpu.sync_copy(ovec, out1.at[pl.ds(wid * L, L)])

    CW = N // NS
    pltpu.sync_copy(colmin, shared.at[pl.ds(s * N, N)])
    plsc.subcore_barrier()
    cps = [pltpu.async_copy(shared.at[pl.ds(k * N + s * CW, CW)],
                            tbuf.at[pl.ds(k * CW, CW)], dsem)
           for k in range(NS)]
    for cp in cps:
        cp.wait()

    def cmb_body(i, carry):
        acc = tbuf[pl.ds(i * L, L)]
        for k in range(1, NS):
            acc = jnp.minimum(acc, tbuf[pl.ds(k * CW + i * L, L)])
        colmin[pl.ds(i * L, L)] = acc
        return carry
    lax.fori_loop(0, CW // L, cmb_body, 0)
    pltpu.sync_copy(colmin.at[pl.ds(0, CW)],
                    outc.at[pl.ds(c * N + s * CW, CW)])


_sc_b0 = functools.partial(
    pl.kernel,
    out_type=[jax.ShapeDtypeStruct((NW * L,), jnp.float32),
              jax.ShapeDtypeStruct((NC * N,), jnp.float32)],
    mesh=plsc.VectorSubcoreMesh(core_axis_name="c", subcore_axis_name="s",
                                num_cores=NC, num_subcores=NS),
    scratch_types=[
        pltpu.VMEM((WQ,), jnp.float32),
        pltpu.VMEM((WQ,), jnp.float32),
        pltpu.VMEM((WQ,), jnp.float32),
        pltpu.VMEM((N,), jnp.float32),
        pltpu.VMEM((N,), jnp.float32),
        pltpu.VMEM((N,), jnp.float32),
        pltpu.VMEM((WQ,), jnp.float32),
        pltpu.VMEM((N,), jnp.float32),
        pltpu.VMEM((N,), jnp.float32),
        pltpu.VMEM((WQ * L,), jnp.float32),
        pltpu.VMEM((N,), jnp.float32),
        pltpu.VMEM((L,), jnp.float32),
        pltpu.VMEM_SHARED((NS * N,), jnp.float32),
        pltpu.SemaphoreType.DMA,
    ],
    compiler_params=pltpu.CompilerParams(needs_layout_passes=False),
)(_sc_body)


def _tc_body(x1e_ref, x2te_ref, x1n_ref, x2n_ref, d1_ref, d2_ref):
    x1e = x1e_ref[0]
    x2te = x2te_ref[0]
    x1s = jnp.sum(x1e * x1e, axis=1, keepdims=True)
    x2s = jnp.sum(x2te * x2te, axis=0, keepdims=True)
    cross2 = jax.lax.dot_general(
        x1n_ref[0], x2n_ref[0], (((1,), (0,)), ((), ())),
        preferred_element_type=jnp.float32)
    d = (x1s + x2s) + cross2
    d1_ref[0, 0] = jnp.min(d, axis=1, keepdims=True)
    d2_ref[0, 0] = jnp.min(d, axis=0, keepdims=True)


_tc_main = pl.pallas_call(
    _tc_body,
    grid=(BT, NBLK),
    in_specs=[
        pl.BlockSpec((1, BLKN, 3), lambda b, n: (b, n, 0)),
        pl.BlockSpec((1, 3, N), lambda b, n: (b, 0, 0)),
        pl.BlockSpec((1, BLKN, 8), lambda b, n: (b, n, 0)),
        pl.BlockSpec((1, 8, N), lambda b, n: (b, 0, 0)),
    ],
    out_specs=[
        pl.BlockSpec((1, 1, BLKN, 1), lambda b, n: (b, n, 0, 0)),
        pl.BlockSpec((1, 1, 1, N), lambda b, n: (b, n, 0, 0)),
    ],
    out_shape=[jax.ShapeDtypeStruct((BT, NBLK, BLKN, 1), jnp.float32),
               jax.ShapeDtypeStruct((BT, NBLK, 1, N), jnp.float32)],
    compiler_params=pltpu.CompilerParams(
        dimension_semantics=("parallel", "parallel")),
)


def _fin_body(sc1_ref, scc_ref, td1_ref, td2_ref, out_ref):
    s_sc1 = jnp.sum(sc1_ref[...])
    s_col = jnp.sum(jnp.min(scc_ref[...], axis=0))
    s_d1 = jnp.sum(td1_ref[...])
    s_d2 = jnp.sum(jnp.min(td2_ref[...], axis=1))
    out_ref[0, 0] = ((s_sc1 + s_col) * 2.0 + s_d1 + s_d2) / (B * N)


_finalize = pl.pallas_call(
    _fin_body,
    out_shape=jax.ShapeDtypeStruct((1, 1), jnp.float32),
    out_specs=pl.BlockSpec(memory_space=pltpu.SMEM),
)


@jax.jit
def kernel(xyz1, xyz2):
    x1t = jnp.transpose(xyz1, (0, 2, 1))
    x2t = jnp.transpose(xyz2, (0, 2, 1))

    x1f0 = x1t[0].reshape(-1)
    x2f0 = x2t[0].reshape(-1)
    x1r0 = lax.reduce_precision(x1t[0], 8, 7).reshape(-1)
    x2r0 = lax.reduce_precision(x2t[0], 8, 7).reshape(-1)

    x1n = jnp.pad((xyz1[1:] * jnp.float32(-2.0)).astype(jnp.bfloat16),
                  ((0, 0), (0, 0), (0, 5)))
    x2n = jnp.pad(x2t[1:].astype(jnp.bfloat16),
                  ((0, 0), (0, 5), (0, 0)))
    x1e = xyz1[1:]
    x2te = x2t[1:]

    sc1, scc = _sc_b0(x1f0, x2f0, x1r0, x2r0)
    td1, td2 = _tc_main(x1e, x2te, x1n, x2n)
    out = _finalize(sc1.reshape(1, NW * L), scc.reshape(NC, N),
                    td1.reshape(BT * NBLK, BLKN), td2.reshape(BT, NBLK, N))
    return out.reshape(())


# speedup vs baseline: 1.0751x; 1.0751x over previous
import functools

import jax
import jax.numpy as jnp
from jax import lax
from jax.experimental import pallas as pl
from jax.experimental.pallas import tpu as pltpu
from jax.experimental.pallas import tpu_sc as plsc

NC, NS, L = 2, 16, 16
B, N, D = 8, 4096, 3
NW = NC * NS
WQ = N // NW
QB = 8
NV = N // L

BT = B - 1
BLKN = 1024
NBLK = N // BLKN


def _sc_body(x1f, x2f, x1r, x2r, out1, outc,
             q1x, q1y, q1z, x2x, x2y, x2z, hq2, hp2, colmin, rowacc,
             tbuf, ovec, shared, dsem):
    c = lax.axis_index("c")
    s = lax.axis_index("s")
    wid = c * NS + s

    qb0 = wid * WQ
    cps = [pltpu.async_copy(x2f.at[pl.ds(0, N)], x2x, dsem),
           pltpu.async_copy(x2f.at[pl.ds(N, N)], x2y, dsem),
           pltpu.async_copy(x2f.at[pl.ds(2 * N, N)], x2z, dsem),
           pltpu.async_copy(x1f.at[pl.ds(qb0, WQ)], q1x, dsem),
           pltpu.async_copy(x1f.at[pl.ds(N + qb0, WQ)], q1y, dsem),
           pltpu.async_copy(x1f.at[pl.ds(2 * N + qb0, WQ)], q1z, dsem)]
    for cp in cps:
        cp.wait()

    half = jnp.float32(0.5)

    def hp_body(j, carry):
        sl = pl.ds(j * L, L)
        px, py, pz = x2x[sl], x2y[sl], x2z[sl]
        hp2[sl] = (px * px + py * py + pz * pz) * half
        return carry
    lax.fori_loop(0, NV, hp_body, 0)

    def hq_body(j, carry):
        sl = pl.ds(j * L, L)
        qx, qy, qz = q1x[sl], q1y[sl], q1z[sl]
        hq2[sl] = (qx * qx + qy * qy + qz * qz) * half
        return carry
    lax.fori_loop(0, WQ // L, hq_body, 0)

    cps = [pltpu.async_copy(x2r.at[pl.ds(0, N)], x2x, dsem),
           pltpu.async_copy(x2r.at[pl.ds(N, N)], x2y, dsem),
           pltpu.async_copy(x2r.at[pl.ds(2 * N, N)], x2z, dsem),
           pltpu.async_copy(x1r.at[pl.ds(qb0, WQ)], q1x, dsem),
           pltpu.async_copy(x1r.at[pl.ds(N + qb0, WQ)], q1y, dsem),
           pltpu.async_copy(x1r.at[pl.ds(2 * N + qb0, WQ)], q1z, dsem)]
    for cp in cps:
        cp.wait()

    inf_v = jnp.full((L,), jnp.inf, jnp.float32)

    def init_body(j, carry):
        colmin[pl.ds(j * L, L)] = inf_v
        return carry
    lax.fori_loop(0, NV, init_body, 0)

    def qg_body(qg, carry):
        sl = pl.ds(qg * L, L)
        qvx, qvy, qvz, qv2 = q1x[sl], q1y[sl], q1z[sl], hq2[sl]
        for h in range(L // QB):
            qx = [jnp.full((L,), qvx[h * QB + i]) for i in range(QB)]
            qy = [jnp.full((L,), qvy[h * QB + i]) for i in range(QB)]
            qz = [jnp.full((L,), qvz[h * QB + i]) for i in range(QB)]
            q2 = [jnp.full((L,), qv2[h * QB + i]) for i in range(QB)]

            def j_body(j, accs):
                jsl = pl.ds(j * L, L)
                px = x2x[jsl]
                py = x2y[jsl]
                pz = x2z[jsl]
                p2 = hp2[jsl]
                ds_ = []
                new_accs = []
                for i in range(QB):
                    cr = px * qx[i] + py * qy[i] + pz * qz[i]
                    d = (p2 + q2[i]) - cr
                    ds_.append(d)
                    new_accs.append(jnp.minimum(accs[i], d))
                m01 = jnp.minimum(ds_[0], ds_[1])
                m23 = jnp.minimum(ds_[2], ds_[3])
                m45 = jnp.minimum(ds_[4], ds_[5])
                m67 = jnp.minimum(ds_[6], ds_[7])
                m = jnp.minimum(jnp.minimum(m01, m23),
                                jnp.minimum(m45, m67))
                colmin[jsl] = jnp.minimum(colmin[jsl], m)
                return tuple(new_accs)

            accs = lax.fori_loop(0, NV, j_body, (inf_v,) * QB)
            for i in range(QB):
                q = qg * L + h * QB + i
                rowacc[pl.ds(q * L, L)] = accs[i]
        return carry

    lax.fori_loop(0, WQ // L, qg_body, 0)

    iota = lax.iota(jnp.int32, L)

    def rg_body(g, s1v):
        idx0 = g * (L * L) + iota * L
        rmin = plsc.load_gather(rowacc, [idx0])
        for j in range(1, L):
            rmin = jnp.minimum(rmin, plsc.load_gather(rowacc, [idx0 + j]))
        return s1v + rmin

    s1v = lax.fori_loop(0, WQ // L, rg_body, jnp.zeros((L,), jnp.float32))
    ovec[pl.ds(0, L)] = s1v
    pltpu.sync_copy(ovec, out1.at[pl.ds(wid * L, L)])

    CW = N // NS
    pltpu.sync_copy(colmin, shared.at[pl.ds(s * N, N)])
    plsc.subcore_barrier()
    cps = [pltpu.async_copy(shared.at[pl.ds(k * N + s * CW, CW)],
                            tbuf.at[pl.ds(k * CW, CW)], dsem)
           for k in range(NS)]
    for cp in cps:
        cp.wait()

    def cmb_body(i, carry):
        acc = tbuf[pl.ds(i * L, L)]
        for k in range(1, NS):
            acc = jnp.minimum(acc, tbuf[pl.ds(k * CW + i * L, L)])
        colmin[pl.ds(i * L, L)] = acc
        return carry
    lax.fori_loop(0, CW // L, cmb_body, 0)
    pltpu.sync_copy(colmin.at[pl.ds(0, CW)],
                    outc.at[pl.ds(c * N + s * CW, CW)])


_sc_b0 = functools.partial(
    pl.kernel,
    out_type=[jax.ShapeDtypeStruct((NW * L,), jnp.float32),
              jax.ShapeDtypeStruct((NC * N,), jnp.float32)],
    mesh=plsc.VectorSubcoreMesh(core_axis_name="c", subcore_axis_name="s",
                                num_cores=NC, num_subcores=NS),
    scratch_types=[
        pltpu.VMEM((WQ,), jnp.float32),
        pltpu.VMEM((WQ,), jnp.float32),
        pltpu.VMEM((WQ,), jnp.float32),
        pltpu.VMEM((N,), jnp.float32),
        pltpu.VMEM((N,), jnp.float32),
        pltpu.VMEM((N,), jnp.float32),
        pltpu.VMEM((WQ,), jnp.float32),
        pltpu.VMEM((N,), jnp.float32),
        pltpu.VMEM((N,), jnp.float32),
        pltpu.VMEM((WQ * L,), jnp.float32),
        pltpu.VMEM((N,), jnp.float32),
        pltpu.VMEM((L,), jnp.float32),
        pltpu.VMEM_SHARED((NS * N,), jnp.float32),
        pltpu.SemaphoreType.DMA,
    ],
    compiler_params=pltpu.CompilerParams(needs_layout_passes=False),
)(_sc_body)


def _tc_body(x1e_ref, x2te_ref, x1n_ref, x2n_ref, d1_ref, d2_ref):
    x1e = x1e_ref[0]
    x2te = x2te_ref[0]
    x1s = jnp.sum(x1e * x1e, axis=1, keepdims=True)
    x2s = jnp.sum(x2te * x2te, axis=0, keepdims=True)
    cross2 = jax.lax.dot_general(
        x1n_ref[0], x2n_ref[0], (((1,), (0,)), ((), ())),
        preferred_element_type=jnp.float32)
    d = (x1s + x2s) + cross2
    d1_ref[0, 0] = jnp.min(d, axis=1, keepdims=True)
    d2_ref[0, 0] = jnp.min(d, axis=0, keepdims=True)


_tc_main = pl.pallas_call(
    _tc_body,
    grid=(BT, NBLK),
    in_specs=[
        pl.BlockSpec((1, BLKN, 3), lambda b, n: (b, n, 0)),
        pl.BlockSpec((1, 3, N), lambda b, n: (b, 0, 0)),
        pl.BlockSpec((1, BLKN, 8), lambda b, n: (b, n, 0)),
        pl.BlockSpec((1, 8, N), lambda b, n: (b, 0, 0)),
    ],
    out_specs=[
        pl.BlockSpec((1, 1, BLKN, 1), lambda b, n: (b, n, 0, 0)),
        pl.BlockSpec((1, 1, 1, N), lambda b, n: (b, n, 0, 0)),
    ],
    out_shape=[jax.ShapeDtypeStruct((BT, NBLK, BLKN, 1), jnp.float32),
               jax.ShapeDtypeStruct((BT, NBLK, 1, N), jnp.float32)],
    compiler_params=pltpu.CompilerParams(
        dimension_semantics=("parallel", "parallel")),
)


def _fin_body(sc1_ref, scc_ref, td1_ref, td2_ref, out_ref):
    s_sc1 = jnp.sum(sc1_ref[...])
    s_col = jnp.sum(jnp.min(scc_ref[...], axis=0))
    s_d1 = jnp.sum(td1_ref[...])
    s_d2 = jnp.sum(jnp.min(td2_ref[...], axis=1))
    out_ref[0, 0] = ((s_sc1 + s_col) * 2.0 + s_d1 + s_d2) / (B * N)


_finalize = pl.pallas_call(
    _fin_body,
    out_shape=jax.ShapeDtypeStruct((1, 1), jnp.float32),
    out_specs=pl.BlockSpec(memory_space=pltpu.SMEM),
)


@jax.jit
def kernel(xyz1, xyz2):
    x1t = jnp.transpose(xyz1, (0, 2, 1))
    x2t = jnp.transpose(xyz2, (0, 2, 1))

    x1f0 = x1t[0].reshape(-1)
    x2f0 = x2t[0].reshape(-1)
    x1r0 = lax.reduce_precision(x1t[0], 8, 7).reshape(-1)
    x2r0 = lax.reduce_precision(x2t[0], 8, 7).reshape(-1)

    x1n = jnp.pad((xyz1[1:] * jnp.float32(-2.0)).astype(jnp.bfloat16),
                  ((0, 0), (0, 0), (0, 5)))
    x2n = jnp.pad(x2t[1:].astype(jnp.bfloat16),
                  ((0, 0), (0, 5), (0, 0)))
    x1e = xyz1[1:]
    x2te = x2t[1:]

    sc1, scc = _sc_b0(x1f0, x2f0, x1r0, x2r0)
    td1, td2 = _tc_main(x1e, x2te, x1n, x2n)
    out = _finalize(sc1.reshape(1, NW * L), scc.reshape(NC, N),
                    td1.reshape(BT * NBLK, BLKN), td2.reshape(BT, NBLK, N))
    return out.reshape(())


# speedup vs baseline: 1.2450x; 1.1580x over previous
import functools

import jax
import jax.numpy as jnp
from jax import lax
from jax.experimental import pallas as pl
from jax.experimental.pallas import tpu as pltpu
from jax.experimental.pallas import tpu_sc as plsc

NC, NS, L = 2, 16, 16
B, N, D = 8, 4096, 3
NW = NC * NS
WQ = N // NW
QB = 8
NV = N // L

BT = B - 1
BLKN = 1024
NBLK = N // BLKN


def _sc_body(x1f, x2f, x1r, x2r, out1, outc,
             q1x, q1y, q1z, x2x, x2y, x2z, hq2, hp2, colmin, rowacc,
             tbuf, ovec, shared, dsem):
    c = lax.axis_index("c")
    s = lax.axis_index("s")
    wid = c * NS + s

    qb0 = wid * WQ
    cps = [pltpu.async_copy(x2f.at[pl.ds(0, N)], x2x, dsem),
           pltpu.async_copy(x2f.at[pl.ds(N, N)], x2y, dsem),
           pltpu.async_copy(x2f.at[pl.ds(2 * N, N)], x2z, dsem),
           pltpu.async_copy(x1f.at[pl.ds(qb0, WQ)], q1x, dsem),
           pltpu.async_copy(x1f.at[pl.ds(N + qb0, WQ)], q1y, dsem),
           pltpu.async_copy(x1f.at[pl.ds(2 * N + qb0, WQ)], q1z, dsem)]
    for cp in cps:
        cp.wait()

    half = jnp.float32(0.5)

    def hp_body(j, carry):
        sl = pl.ds(j * L, L)
        px, py, pz = x2x[sl], x2y[sl], x2z[sl]
        hp2[sl] = (px * px + py * py + pz * pz) * half
        return carry
    lax.fori_loop(0, NV, hp_body, 0)

    def hq_body(j, carry):
        sl = pl.ds(j * L, L)
        qx, qy, qz = q1x[sl], q1y[sl], q1z[sl]
        hq2[sl] = (qx * qx + qy * qy + qz * qz) * half
        return carry
    lax.fori_loop(0, WQ // L, hq_body, 0)

    cps = [pltpu.async_copy(x2r.at[pl.ds(0, N)], x2x, dsem),
           pltpu.async_copy(x2r.at[pl.ds(N, N)], x2y, dsem),
           pltpu.async_copy(x2r.at[pl.ds(2 * N, N)], x2z, dsem),
           pltpu.async_copy(x1r.at[pl.ds(qb0, WQ)], q1x, dsem),
           pltpu.async_copy(x1r.at[pl.ds(N + qb0, WQ)], q1y, dsem),
           pltpu.async_copy(x1r.at[pl.ds(2 * N + qb0, WQ)], q1z, dsem)]
    for cp in cps:
        cp.wait()

    inf_v = jnp.full((L,), jnp.inf, jnp.float32)

    def init_body(j, carry):
        colmin[pl.ds(j * L, L)] = inf_v
        return carry
    lax.fori_loop(0, NV, init_body, 0)

    def qg_body(qg, carry):
        sl = pl.ds(qg * L, L)
        qvx, qvy, qvz, qv2 = q1x[sl], q1y[sl], q1z[sl], hq2[sl]
        for h in range(L // QB):
            qx = [jnp.full((L,), qvx[h * QB + i]) for i in range(QB)]
            qy = [jnp.full((L,), qvy[h * QB + i]) for i in range(QB)]
            qz = [jnp.full((L,), qvz[h * QB + i]) for i in range(QB)]
            q2 = [jnp.full((L,), qv2[h * QB + i]) for i in range(QB)]

            def j_body(j, accs):
                jsl = pl.ds(j * L, L)
                px = x2x[jsl]
                py = x2y[jsl]
                pz = x2z[jsl]
                p2 = hp2[jsl]
                ds_ = []
                new_accs = []
                for i in range(QB):
                    cr = px * qx[i] + py * qy[i] + pz * qz[i]
                    d = (p2 + q2[i]) - cr
                    ds_.append(d)
                    new_accs.append(jnp.minimum(accs[i], d))
                m01 = jnp.minimum(ds_[0], ds_[1])
                m23 = jnp.minimum(ds_[2], ds_[3])
                m45 = jnp.minimum(ds_[4], ds_[5])
                m67 = jnp.minimum(ds_[6], ds_[7])
                m = jnp.minimum(jnp.minimum(m01, m23),
                                jnp.minimum(m45, m67))
                colmin[jsl] = jnp.minimum(colmin[jsl], m)
                return tuple(new_accs)

            accs = lax.fori_loop(0, NV, j_body, (inf_v,) * QB)
            for i in range(QB):
                q = qg * L + h * QB + i
                rowacc[pl.ds(q * L, L)] = accs[i]
        return carry

    lax.fori_loop(0, WQ // L, qg_body, 0)

    iota = lax.iota(jnp.int32, L)

    def rg_body(g, s1v):
        idx0 = g * (L * L) + iota * L
        rmin = plsc.load_gather(rowacc, [idx0])
        for j in range(1, L):
            rmin = jnp.minimum(rmin, plsc.load_gather(rowacc, [idx0 + j]))
        return s1v + rmin

    s1v = lax.fori_loop(0, WQ // L, rg_body, jnp.zeros((L,), jnp.float32))
    ovec[pl.ds(0, L)] = s1v
    pltpu.sync_copy(ovec, out1.at[pl.ds(wid * L, L)])

    CW = N // NS
    pltpu.sync_copy(colmin, shared.at[pl.ds(s * N, N)])
    plsc.subcore_barrier()
    cps = [pltpu.async_copy(shared.at[pl.ds(k * N + s * CW, CW)],
                            tbuf.at[pl.ds(k * CW, CW)], dsem)
           for k in range(NS)]
    for cp in cps:
        cp.wait()

    def cmb_body(i, carry):
        acc = tbuf[pl.ds(i * L, L)]
        for k in range(1, NS):
            acc = jnp.minimum(acc, tbuf[pl.ds(k * CW + i * L, L)])
        colmin[pl.ds(i * L, L)] = acc
        return carry
    lax.fori_loop(0, CW // L, cmb_body, 0)
    pltpu.sync_copy(colmin.at[pl.ds(0, CW)],
                    outc.at[pl.ds(c * N + s * CW, CW)])


_sc_b0 = functools.partial(
    pl.kernel,
    out_type=[jax.ShapeDtypeStruct((NW * L,), jnp.float32),
              jax.ShapeDtypeStruct((NC * N,), jnp.float32)],
    mesh=plsc.VectorSubcoreMesh(core_axis_name="c", subcore_axis_name="s",
                                num_cores=NC, num_subcores=NS),
    scratch_types=[
        pltpu.VMEM((WQ,), jnp.float32),
        pltpu.VMEM((WQ,), jnp.float32),
        pltpu.VMEM((WQ,), jnp.float32),
        pltpu.VMEM((N,), jnp.float32),
        pltpu.VMEM((N,), jnp.float32),
        pltpu.VMEM((N,), jnp.float32),
        pltpu.VMEM((WQ,), jnp.float32),
        pltpu.VMEM((N,), jnp.float32),
        pltpu.VMEM((N,), jnp.float32),
        pltpu.VMEM((WQ * L,), jnp.float32),
        pltpu.VMEM((N,), jnp.float32),
        pltpu.VMEM((L,), jnp.float32),
        pltpu.VMEM_SHARED((NS * N,), jnp.float32),
        pltpu.SemaphoreType.DMA,
    ],
    compiler_params=pltpu.CompilerParams(needs_layout_passes=False),
)(_sc_body)


def _tc_body(x1e_ref, x2te_ref, d1_ref, d2_ref):
    x1e = x1e_ref[0]
    x2te = x2te_ref[0]
    x1s = jnp.sum(x1e * x1e, axis=1, keepdims=True)
    x2s = jnp.sum(x2te * x2te, axis=0, keepdims=True)
    x1b = (x1e * jnp.float32(-2.0)).astype(jnp.bfloat16)
    x2b = x2te.astype(jnp.bfloat16)
    cross2 = jax.lax.dot_general(
        x1b, x2b, (((1,), (0,)), ((), ())),
        preferred_element_type=jnp.float32)
    d = (x1s + x2s) + cross2
    d1_ref[0, 0] = jnp.min(d, axis=1, keepdims=True)
    d2_ref[0, 0] = jnp.min(d, axis=0, keepdims=True)


_tc_main = pl.pallas_call(
    _tc_body,
    grid=(BT, NBLK),
    in_specs=[
        pl.BlockSpec((1, BLKN, 3), lambda b, n: (b, n, 0)),
        pl.BlockSpec((1, 3, N), lambda b, n: (b, 0, 0)),
    ],
    out_specs=[
        pl.BlockSpec((1, 1, BLKN, 1), lambda b, n: (b, n, 0, 0)),
        pl.BlockSpec((1, 1, 1, N), lambda b, n: (b, n, 0, 0)),
    ],
    out_shape=[jax.ShapeDtypeStruct((BT, NBLK, BLKN, 1), jnp.float32),
               jax.ShapeDtypeStruct((BT, NBLK, 1, N), jnp.float32)],
    compiler_params=pltpu.CompilerParams(
        dimension_semantics=("parallel", "parallel")),
)


def _fin_body(sc1_ref, scc_ref, td1_ref, td2_ref, out_ref):
    s_sc1 = jnp.sum(sc1_ref[...])
    s_col = jnp.sum(jnp.min(scc_ref[...], axis=0))
    s_d1 = jnp.sum(td1_ref[...])
    s_d2 = jnp.sum(jnp.min(td2_ref[...], axis=1))
    out_ref[0, 0] = ((s_sc1 + s_col) * 2.0 + s_d1 + s_d2) / (B * N)


_finalize = pl.pallas_call(
    _fin_body,
    out_shape=jax.ShapeDtypeStruct((1, 1), jnp.float32),
    out_specs=pl.BlockSpec(memory_space=pltpu.SMEM),
)


@jax.jit
def kernel(xyz1, xyz2):
    x1t = jnp.transpose(xyz1, (0, 2, 1))
    x2t = jnp.transpose(xyz2, (0, 2, 1))

    x1f0 = x1t[0].reshape(-1)
    x2f0 = x2t[0].reshape(-1)
    x1r0 = lax.reduce_precision(x1t[0], 8, 7).reshape(-1)
    x2r0 = lax.reduce_precision(x2t[0], 8, 7).reshape(-1)

    x1e = xyz1[1:]
    x2te = x2t[1:]

    sc1, scc = _sc_b0(x1f0, x2f0, x1r0, x2r0)
    td1, td2 = _tc_main(x1e, x2te)
    out = _finalize(sc1.reshape(1, NW * L), scc.reshape(NC, N),
                    td1.reshape(BT * NBLK, BLKN), td2.reshape(BT, NBLK, N))
    return out.reshape(())
